# baseline (speedup 1.0000x reference)
import functools

import jax
import jax.numpy as jnp
from jax import lax
from jax.experimental import pallas as pl
from jax.experimental.pallas import tpu as pltpu
from jax.experimental.pallas import tpu_sc as plsc

BATCH = 16384
DIM = 32
NROW = 1000000
NC = 2
NS = 16
NW = NC * NS
BPW = BATCH // NW
CHUNK = 128
NCHUNK = BPW // CHUNK
GRP = 16


def _body(ut_hbm, it_hbm, uidx8_hbm, ulane_hbm, iidx8_hbm, ilane_hbm, out_hbm,
          uidx8_v, ulane_v, iidx8_v, ilane_v, ubuf, ibuf, outc, sem_u, sem_i):
    wid = lax.axis_index("s") * NC + lax.axis_index("c")
    base = wid * BPW

    pltpu.sync_copy(uidx8_hbm.at[pl.ds(wid * NCHUNK, NCHUNK)], uidx8_v)
    pltpu.sync_copy(ulane_hbm.at[pl.ds(wid * NCHUNK, NCHUNK)], ulane_v)
    pltpu.sync_copy(iidx8_hbm.at[pl.ds(wid * NCHUNK, NCHUNK)], iidx8_v)
    pltpu.sync_copy(ilane_hbm.at[pl.ds(wid * NCHUNK, NCHUNK)], ilane_v)

    for c in range(NCHUNK):
        def fire(d, _):
            pltpu.async_copy(
                ut_hbm.at[d].at[uidx8_v.at[c]], ubuf.at[d], sem_u)
            pltpu.async_copy(
                it_hbm.at[d].at[iidx8_v.at[c]], ibuf.at[d], sem_i)
            return 0

        lax.fori_loop(0, DIM, fire, 0)
        pltpu.make_async_copy(
            ut_hbm.at[:, pl.ds(0, CHUNK), :], ubuf, sem_u).wait()
        pltpu.make_async_copy(
            it_hbm.at[:, pl.ds(0, CHUNK), :], ibuf, sem_i).wait()

        def group(g, _):
            off = pl.multiple_of(g * GRP, GRP)
            rows = jnp.full((GRP,), 0, jnp.int32) + off + lax.iota(
                jnp.int32, GRP)
            ul = ulane_v[c, pl.ds(off, GRP)]
            il = ilane_v[c, pl.ds(off, GRP)]
            acc = jnp.zeros((GRP,), jnp.float32)
            for d in range(DIM):
                dcol = jnp.full((GRP,), d, jnp.int32)
                uv = plsc.load_gather(ubuf, [dcol, rows, ul])
                iv = plsc.load_gather(ibuf, [dcol, rows, il])
                acc = acc + uv * iv
            outc[pl.ds(off, GRP)] = acc
            return 0

        lax.fori_loop(0, CHUNK // GRP, group, 0)
        pltpu.sync_copy(outc, out_hbm.at[pl.ds(base + c * CHUNK, CHUNK)])


@jax.jit
def _cf_dot(ut, it, uidx8, ulane, iidx8, ilane):
    mesh = plsc.VectorSubcoreMesh(core_axis_name="c", subcore_axis_name="s")
    k = functools.partial(
        pl.kernel,
        mesh=mesh,
        out_type=jax.ShapeDtypeStruct((BATCH,), jnp.float32),
        scratch_types=[
            pltpu.VMEM((NCHUNK, CHUNK), jnp.int32),
            pltpu.VMEM((NCHUNK, CHUNK), jnp.int32),
            pltpu.VMEM((NCHUNK, CHUNK), jnp.int32),
            pltpu.VMEM((NCHUNK, CHUNK), jnp.int32),
            pltpu.VMEM((DIM, CHUNK, 8), jnp.float32),
            pltpu.VMEM((DIM, CHUNK, 8), jnp.float32),
            pltpu.VMEM((CHUNK,), jnp.float32),
            pltpu.SemaphoreType.DMA,
            pltpu.SemaphoreType.DMA,
        ],
        compiler_params=pltpu.CompilerParams(
            needs_layout_passes=False, use_tc_tiling_on_sc=False),
    )(_body)
    return k(ut, it, uidx8, ulane, iidx8, ilane)


def kernel(x, user_emb, item_emb):
    x32 = x.astype(jnp.int32)
    uidx = x32[:, 0]
    iidx = x32[:, 1]
    uidx8 = (uidx // 8).reshape(BATCH // CHUNK, CHUNK)
    ulane = (uidx % 8).reshape(BATCH // CHUNK, CHUNK)
    iidx8 = (iidx // 8).reshape(BATCH // CHUNK, CHUNK)
    ilane = (iidx % 8).reshape(BATCH // CHUNK, CHUNK)
    ut = user_emb.T.reshape(DIM, NROW // 8, 8)
    it = item_emb.T.reshape(DIM, NROW // 8, 8)
    return _cf_dot(ut, it, uidx8, ulane, iidx8, ilane)


# speedup vs baseline: 17.1158x; 17.1158x over previous
import functools

import jax
import jax.numpy as jnp
from jax import lax
from jax.experimental import pallas as pl
from jax.experimental.pallas import tpu as pltpu
from jax.experimental.pallas import tpu_sc as plsc

BATCH = 16384
DIM = 32
LTILE = 128
NC = 2
NS = 16
NW = NC * NS
BPW = BATCH // NW
GRP = 16
NSLOT = 8
NGRP = BPW // GRP


def _body(ut_hbm, it_hbm, uct_hbm, ulan_hbm, ict_hbm, ilan_hbm, out_hbm,
          uct_v, ulan_v, ict_v, ilan_v, wu, wi, uex, iex, outv, sem_u, sem_i):
    wid = lax.axis_index("s") * NC + lax.axis_index("c")
    base = wid * BPW

    pltpu.sync_copy(uct_hbm.at[pl.ds(base, BPW)], uct_v)
    pltpu.sync_copy(ulan_hbm.at[pl.ds(base, BPW)], ulan_v)
    pltpu.sync_copy(ict_hbm.at[pl.ds(base, BPW)], ict_v)
    pltpu.sync_copy(ilan_hbm.at[pl.ds(base, BPW)], ilan_v)

    d_lo = lax.iota(jnp.int32, GRP)
    d_hi = d_lo + GRP

    def group(g, _):
        off = pl.multiple_of(g * GRP, GRP)
        uct = uct_v[pl.ds(off, GRP)]
        ulan = ulan_v[pl.ds(off, GRP)]
        ict = ict_v[pl.ds(off, GRP)]
        ilan = ilan_v[pl.ds(off, GRP)]

        for half in range(2):
            for j in range(NSLOT):
                p = half * NSLOT + j
                ou = pl.multiple_of(uct[p] * LTILE, LTILE)
                pltpu.async_copy(
                    ut_hbm.at[:, pl.ds(ou, LTILE)], wu.at[j], sem_u)
                oi = pl.multiple_of(ict[p] * LTILE, LTILE)
                pltpu.async_copy(
                    it_hbm.at[:, pl.ds(oi, LTILE)], wi.at[j], sem_i)
            for j in range(NSLOT):
                pltpu.make_async_copy(
                    ut_hbm.at[:, pl.ds(0, LTILE)], wu.at[j], sem_u).wait()
                pltpu.make_async_copy(
                    it_hbm.at[:, pl.ds(0, LTILE)], wi.at[j], sem_i).wait()
            for j in range(NSLOT):
                p = half * NSLOT + j
                slot = jnp.full((GRP,), j, jnp.int32)
                col = jnp.full((GRP,), p, jnp.int32)
                ul = jnp.full((GRP,), ulan[p], jnp.int32)
                plsc.store_scatter(
                    uex, [d_lo, col], plsc.load_gather(wu, [slot, d_lo, ul]))
                plsc.store_scatter(
                    uex, [d_hi, col], plsc.load_gather(wu, [slot, d_hi, ul]))
                il = jnp.full((GRP,), ilan[p], jnp.int32)
                plsc.store_scatter(
                    iex, [d_lo, col], plsc.load_gather(wi, [slot, d_lo, il]))
                plsc.store_scatter(
                    iex, [d_hi, col], plsc.load_gather(wi, [slot, d_hi, il]))

        acc = jnp.zeros((GRP,), jnp.float32)
        for d in range(DIM):
            acc = acc + uex[d, pl.ds(0, GRP)] * iex[d, pl.ds(0, GRP)]
        outv[pl.ds(off, GRP)] = acc
        return 0

    lax.fori_loop(0, NGRP, group, 0)
    pltpu.sync_copy(outv, out_hbm.at[pl.ds(base, BPW)])


@jax.jit
def _cf_dot(ut, it, uct, ulan, ict, ilan):
    mesh = plsc.VectorSubcoreMesh(core_axis_name="c", subcore_axis_name="s")
    k = functools.partial(
        pl.kernel,
        mesh=mesh,
        out_type=jax.ShapeDtypeStruct((BATCH,), jnp.float32),
        scratch_types=[
            pltpu.VMEM((BPW,), jnp.int32),
            pltpu.VMEM((BPW,), jnp.int32),
            pltpu.VMEM((BPW,), jnp.int32),
            pltpu.VMEM((BPW,), jnp.int32),
            pltpu.VMEM((NSLOT, DIM, LTILE), jnp.float32),
            pltpu.VMEM((NSLOT, DIM, LTILE), jnp.float32),
            pltpu.VMEM((DIM, GRP), jnp.float32),
            pltpu.VMEM((DIM, GRP), jnp.float32),
            pltpu.VMEM((BPW,), jnp.float32),
            pltpu.SemaphoreType.DMA,
            pltpu.SemaphoreType.DMA,
        ],
        compiler_params=pltpu.CompilerParams(
            needs_layout_passes=False, use_tc_tiling_on_sc=True),
    )(_body)
    return k(ut, it, uct, ulan, ict, ilan)


def kernel(x, user_emb, item_emb):
    x32 = x.astype(jnp.int32)
    uidx = x32[:, 0]
    iidx = x32[:, 1]
    return _cf_dot(user_emb.T, item_emb.T,
                   uidx // LTILE, uidx % LTILE, iidx // LTILE, iidx % LTILE)


# speedup vs baseline: 19.0305x; 1.1119x over previous
import functools

import jax
import jax.numpy as jnp
from jax import lax
from jax.experimental import pallas as pl
from jax.experimental.pallas import tpu as pltpu
from jax.experimental.pallas import tpu_sc as plsc

BATCH = 16384
DIM = 32
LTILE = 128
NC = 2
NS = 16
NW = NC * NS
BPW = BATCH // NW
GRP = 16
NSLOT = 8
NGRP = BPW // GRP


def _body(ut_hbm, it_hbm, uct_hbm, ulan_hbm, ict_hbm, ilan_hbm, out_hbm,
          uct_v, ulan_v, ict_v, ilan_v, wu, wi, uex, iex, outv, sem_u, sem_i):
    wid = lax.axis_index("s") * NC + lax.axis_index("c")
    base = wid * BPW

    pltpu.sync_copy(uct_hbm.at[pl.ds(base, BPW)], uct_v)
    pltpu.sync_copy(ulan_hbm.at[pl.ds(base, BPW)], ulan_v)
    pltpu.sync_copy(ict_hbm.at[pl.ds(base, BPW)], ict_v)
    pltpu.sync_copy(ilan_hbm.at[pl.ds(base, BPW)], ilan_v)

    d_lo = lax.iota(jnp.int32, GRP)
    d_hi = d_lo + GRP

    def load_cts(g):
        off = pl.multiple_of(g * GRP, GRP)
        return (uct_v[pl.ds(off, GRP)], ulan_v[pl.ds(off, GRP)],
                ict_v[pl.ds(off, GRP)], ilan_v[pl.ds(off, GRP)])

    def fire(uct, ict, p, j):
        ou = pl.multiple_of(uct[p] * LTILE, LTILE)
        pltpu.async_copy(ut_hbm.at[:, pl.ds(ou, LTILE)], wu.at[j], sem_u)
        oi = pl.multiple_of(ict[p] * LTILE, LTILE)
        pltpu.async_copy(it_hbm.at[:, pl.ds(oi, LTILE)], wi.at[j], sem_i)

    def drain():
        for j in range(NSLOT):
            pltpu.make_async_copy(
                ut_hbm.at[:, pl.ds(0, LTILE)], wu.at[j], sem_u).wait()
            pltpu.make_async_copy(
                it_hbm.at[:, pl.ds(0, LTILE)], wi.at[j], sem_i).wait()

    def extract(ulan, ilan, p, j):
        slot = jnp.full((GRP,), j, jnp.int32)
        col = jnp.full((GRP,), p, jnp.int32)
        ul = jnp.full((GRP,), ulan[p], jnp.int32)
        plsc.store_scatter(
            uex, [d_lo, col], plsc.load_gather(wu, [slot, d_lo, ul]))
        plsc.store_scatter(
            uex, [d_hi, col], plsc.load_gather(wu, [slot, d_hi, ul]))
        il = jnp.full((GRP,), ilan[p], jnp.int32)
        plsc.store_scatter(
            iex, [d_lo, col], plsc.load_gather(wi, [slot, d_lo, il]))
        plsc.store_scatter(
            iex, [d_hi, col], plsc.load_gather(wi, [slot, d_hi, il]))

    uct0, _, ict0, _ = load_cts(0)
    for j in range(NSLOT):
        fire(uct0, ict0, j, j)

    def group(g, _):
        uct, ulan, ict, ilan = load_cts(g)
        gn = lax.min(g + 1, NGRP - 1)
        uctn, _, ictn, _ = load_cts(gn)

        drain()
        for j in range(NSLOT):
            extract(ulan, ilan, j, j)
            fire(uct, ict, NSLOT + j, j)
        drain()
        for j in range(NSLOT):
            extract(ulan, ilan, NSLOT + j, j)
            fire(uctn, ictn, j, j)

        acc = jnp.zeros((GRP,), jnp.float32)
        for d in range(DIM):
            acc = acc + uex[d, pl.ds(0, GRP)] * iex[d, pl.ds(0, GRP)]
        outv[pl.ds(pl.multiple_of(g * GRP, GRP), GRP)] = acc
        return 0

    lax.fori_loop(0, NGRP, group, 0)
    drain()
    pltpu.sync_copy(outv, out_hbm.at[pl.ds(base, BPW)])


@jax.jit
def _cf_dot(ut, it, uct, ulan, ict, ilan):
    mesh = plsc.VectorSubcoreMesh(core_axis_name="c", subcore_axis_name="s")
    k = functools.partial(
        pl.kernel,
        mesh=mesh,
        out_type=jax.ShapeDtypeStruct((BATCH,), jnp.float32),
        scratch_types=[
            pltpu.VMEM((BPW,), jnp.int32),
            pltpu.VMEM((BPW,), jnp.int32),
            pltpu.VMEM((BPW,), jnp.int32),
            pltpu.VMEM((BPW,), jnp.int32),
            pltpu.VMEM((NSLOT, DIM, LTILE), jnp.float32),
            pltpu.VMEM((NSLOT, DIM, LTILE), jnp.float32),
            pltpu.VMEM((DIM, GRP), jnp.float32),
            pltpu.VMEM((DIM, GRP), jnp.float32),
            pltpu.VMEM((BPW,), jnp.float32),
            pltpu.SemaphoreType.DMA,
            pltpu.SemaphoreType.DMA,
        ],
        compiler_params=pltpu.CompilerParams(
            needs_layout_passes=False, use_tc_tiling_on_sc=True),
    )(_body)
    return k(ut, it, uct, ulan, ict, ilan)


def kernel(x, user_emb, item_emb):
    x32 = x.astype(jnp.int32)
    uidx = x32[:, 0]
    iidx = x32[:, 1]
    return _cf_dot(user_emb.T, item_emb.T,
                   uidx // LTILE, uidx % LTILE, iidx // LTILE, iidx % LTILE)


# speedup vs baseline: 19.1358x; 1.0055x over previous
import functools

import jax
import jax.numpy as jnp
from jax import lax
from jax.experimental import pallas as pl
from jax.experimental.pallas import tpu as pltpu
from jax.experimental.pallas import tpu_sc as plsc

BATCH = 16384
DIM = 32
LTILE = 128
NC = 2
NS = 16
NW = NC * NS
BPW = BATCH // NW
GRP = 16
NSLOT = 8
NGRP = BPW // GRP


def _body(ut_hbm, it_hbm, xt_hbm, out_hbm,
          x_v, wu, wi, uex, iex, outv, sem_u, sem_i):
    wid = lax.axis_index("s") * NC + lax.axis_index("c")
    base = wid * BPW

    pltpu.sync_copy(xt_hbm.at[:, pl.ds(base, BPW)], x_v)

    d_lo = lax.iota(jnp.int32, GRP)
    d_hi = d_lo + GRP

    def load_cts(g):
        off = pl.multiple_of(g * GRP, GRP)
        uv = x_v[0, pl.ds(off, GRP)]
        iv = x_v[1, pl.ds(off, GRP)]
        return (uv >> 7, uv & 127, iv >> 7, iv & 127)

    def fire(uct, ict, p, j):
        ou = pl.multiple_of(uct[p] * LTILE, LTILE)
        pltpu.async_copy(ut_hbm.at[:, pl.ds(ou, LTILE)], wu.at[j], sem_u)
        oi = pl.multiple_of(ict[p] * LTILE, LTILE)
        pltpu.async_copy(it_hbm.at[:, pl.ds(oi, LTILE)], wi.at[j], sem_i)

    def drain():
        for j in range(NSLOT):
            pltpu.make_async_copy(
                ut_hbm.at[:, pl.ds(0, LTILE)], wu.at[j], sem_u).wait()
            pltpu.make_async_copy(
                it_hbm.at[:, pl.ds(0, LTILE)], wi.at[j], sem_i).wait()

    def extract(ulan, ilan, p, j):
        slot = jnp.full((GRP,), j, jnp.int32)
        col = jnp.full((GRP,), p, jnp.int32)
        ul = jnp.full((GRP,), ulan[p], jnp.int32)
        plsc.store_scatter(
            uex, [d_lo, col], plsc.load_gather(wu, [slot, d_lo, ul]))
        plsc.store_scatter(
            uex, [d_hi, col], plsc.load_gather(wu, [slot, d_hi, ul]))
        il = jnp.full((GRP,), ilan[p], jnp.int32)
        plsc.store_scatter(
            iex, [d_lo, col], plsc.load_gather(wi, [slot, d_lo, il]))
        plsc.store_scatter(
            iex, [d_hi, col], plsc.load_gather(wi, [slot, d_hi, il]))

    uct0, _, ict0, _ = load_cts(0)
    for j in range(NSLOT):
        fire(uct0, ict0, j, j)

    def group(g, _):
        uct, ulan, ict, ilan = load_cts(g)
        gn = lax.min(g + 1, NGRP - 1)
        uctn, _, ictn, _ = load_cts(gn)

        drain()
        for j in range(NSLOT):
            extract(ulan, ilan, j, j)
            fire(uct, ict, NSLOT + j, j)
        drain()
        for j in range(NSLOT):
            extract(ulan, ilan, NSLOT + j, j)
            fire(uctn, ictn, j, j)

        acc = jnp.zeros((GRP,), jnp.float32)
        for d in range(DIM):
            acc = acc + uex[d, pl.ds(0, GRP)] * iex[d, pl.ds(0, GRP)]
        outv[pl.ds(pl.multiple_of(g * GRP, GRP), GRP)] = acc
        return 0

    lax.fori_loop(0, NGRP, group, 0)
    drain()
    pltpu.sync_copy(outv, out_hbm.at[pl.ds(base, BPW)])


@jax.jit
def _cf_dot(ut, it, xt):
    mesh = plsc.VectorSubcoreMesh(core_axis_name="c", subcore_axis_name="s")
    k = functools.partial(
        pl.kernel,
        mesh=mesh,
        out_type=jax.ShapeDtypeStruct((BATCH,), jnp.float32),
        scratch_types=[
            pltpu.VMEM((2, BPW), jnp.int32),
            pltpu.VMEM((NSLOT, DIM, LTILE), jnp.float32),
            pltpu.VMEM((NSLOT, DIM, LTILE), jnp.float32),
            pltpu.VMEM((DIM, GRP), jnp.float32),
            pltpu.VMEM((DIM, GRP), jnp.float32),
            pltpu.VMEM((BPW,), jnp.float32),
            pltpu.SemaphoreType.DMA,
            pltpu.SemaphoreType.DMA,
        ],
        compiler_params=pltpu.CompilerParams(
            needs_layout_passes=False, use_tc_tiling_on_sc=True),
    )(_body)
    return k(ut, it, xt)


def kernel(x, user_emb, item_emb):
    xt = x.astype(jnp.int32).T
    return _cf_dot(user_emb.T, item_emb.T, xt)
